# Initial kernel scaffold; baseline (speedup 1.0000x reference)
import jax
import jax.numpy as jnp
from jax.experimental import pallas as pl


def kernel(x, edge_index, edge_attr, batch, c0_edge_W, c0_edge_b, c0_pre_W, c0_pre_b, c0_post_W, c0_post_b, c0_lin_W, c0_lin_b, bn0_g, bn0_b, c1_edge_W, c1_edge_b, c1_pre_W, c1_pre_b, c1_post_W, c1_post_b, c1_lin_W, c1_lin_b, bn1_g, bn1_b, mlp_W1, mlp_b1, mlp_W2, mlp_b2, mlp_W3, mlp_b3):
    raise NotImplementedError("write your pallas kernel here")


# speedup vs baseline: 179.6707x; 179.6707x over previous
import functools

import jax
import jax.numpy as jnp
import numpy as np
from jax import lax
from jax.experimental import pallas as pl
from jax.experimental.pallas import tpu as pltpu, tpu_sc as plsc

N = 100000
E = 1600000
NGRAPHS = 512
T = 5
F = 5
NPAD = 100352
NT = NPAD // 32
NGRP = NT // 16
NB = NPAD // 1024
EW = E // 32
WIN = 2000
NWIN = EW // WIN
G = 408
EPAD = E + 512

_DEG_HIST = np.array([0] * 16 + [100000], dtype=np.float64)
_bins = np.arange(len(_DEG_HIST), dtype=np.float64)
AVG_LOG = float((np.log(_bins + 1.0) * _DEG_HIST).sum() / _DEG_HIST.sum())

_SC_MESH = dict(core_axis_name="c", subcore_axis_name="s")
_SC_PARAMS = pltpu.CompilerParams(needs_layout_passes=False)


def _wid():
    return lax.axis_index("s") * 2 + lax.axis_index("c")


def _al8(x):
    return pl.multiple_of(x, 8)


@functools.partial(
    pl.kernel,
    mesh=plsc.VectorSubcoreMesh(**_SC_MESH),
    compiler_params=_SC_PARAMS,
    out_type=jax.ShapeDtypeStruct((32 * NPAD,), jnp.int32),
    scratch_types=[
        pltpu.VMEM((NPAD,), jnp.int32),
        pltpu.VMEM((WIN,), jnp.int32),
    ],
)
def _sc_hist(dst_hbm, hists_hbm, hist_v, dst_v):
    wid = _wid()
    iota = lax.iota(jnp.int32, 16)

    def zero(i, c):
        hist_v[pl.ds(i * 16, 16)] = jnp.zeros((16,), jnp.int32)
        return c

    lax.fori_loop(0, NPAD // 16, zero, 0)

    def window(w, c):
        pltpu.sync_copy(dst_hbm.at[pl.ds(_al8(wid * EW + w * WIN), WIN)], dst_v)

        def blk(i, c2):
            d16 = dst_v[pl.ds(i * 16, 16)]
            sk, _ = plsc.sort_key_val(d16, iota)
            prev = sk.at[jnp.maximum(iota - 1, 0)].get(mode="promise_in_bounds")
            head = (sk != prev) | (iota == 0)
            hpos = plsc.cummax(jnp.where(head, iota, -1))
            rank = iota - hpos
            nxt = sk.at[jnp.minimum(iota + 1, 15)].get(mode="promise_in_bounds")
            tail = (sk != nxt) | (iota == 15)
            plsc.addupdate_scatter(hist_v, [sk], rank + 1, mask=tail)
            return c2

        lax.fori_loop(0, WIN // 16, blk, 0)
        return c

    lax.fori_loop(0, NWIN, window, 0)
    pltpu.sync_copy(hist_v, hists_hbm.at[pl.ds(_al8(wid * NPAD), NPAD)])


def _make_sc_colsum():
    @functools.partial(
        pl.kernel,
        mesh=plsc.VectorSubcoreMesh(**_SC_MESH),
        compiler_params=_SC_PARAMS,
        out_type=[
            jax.ShapeDtypeStruct((NPAD,), jnp.int32),
            jax.ShapeDtypeStruct((NPAD,), jnp.float32),
            jax.ShapeDtypeStruct((512,), jnp.int32),
        ],
        scratch_types=[
            pltpu.VMEM((32 * NT,), jnp.int32),
            pltpu.VMEM((NT,), jnp.int32),
            pltpu.VMEM((NT,), jnp.float32),
            pltpu.VMEM((16,), jnp.int32),
        ],
    )
    def k(hists_hbm, tots_hbm, cntf_hbm, ss_hbm, hcol_v, tot_v, cntf_v, ssst_v):
        wid = _wid()
        for t in range(32):
            pltpu.sync_copy(
                hists_hbm.at[pl.ds(_al8(t * NPAD + wid * NT), NT)],
                hcol_v.at[pl.ds(t * NT, NT)],
            )

        def body(i, part):
            s = hcol_v[pl.ds(i * 16, 16)]
            for t in range(1, 32):
                s = s + hcol_v[pl.ds(t * NT + i * 16, 16)]
            tot_v[pl.ds(i * 16, 16)] = s
            cntf_v[pl.ds(i * 16, 16)] = s.astype(jnp.float32)
            return part + s

        part = lax.fori_loop(0, NGRP, body, jnp.zeros((16,), jnp.int32))
        ssst_v[pl.ds(0, 16)] = part
        pltpu.sync_copy(tot_v, tots_hbm.at[pl.ds(_al8(wid * NT), NT)])
        pltpu.sync_copy(cntf_v, cntf_hbm.at[pl.ds(_al8(wid * NT), NT)])
        pltpu.sync_copy(ssst_v, ss_hbm.at[pl.ds(_al8(wid * 16), 16)])

    return k


_sc_colsum = _make_sc_colsum()


def _make_sc_offsets():
    @functools.partial(
        pl.kernel,
        mesh=plsc.VectorSubcoreMesh(**_SC_MESH),
        compiler_params=_SC_PARAMS,
        out_type=[
            jax.ShapeDtypeStruct((NPAD + 16,), jnp.int32),
            jax.ShapeDtypeStruct((32 * NPAD,), jnp.int32),
        ],
        scratch_types=[
            pltpu.VMEM((512,), jnp.int32),
            pltpu.VMEM((NT,), jnp.int32),
            pltpu.VMEM((NT,), jnp.int32),
            pltpu.VMEM((NT,), jnp.int32),
            pltpu.VMEM((NT,), jnp.int32),
            pltpu.VMEM((16,), jnp.int32),
        ],
    )
    def k(ss_hbm, tots_hbm, hists_hbm, pfx_hbm, starts_hbm,
          ssv, totv, pfxv, runv, hrow, padv):
        wid = _wid()
        pltpu.sync_copy(ss_hbm, ssv)
        carry = jnp.int32(0)
        for t in range(32):
            s_t = jnp.sum(ssv[pl.ds(t * 16, 16)])
            carry = carry + jnp.where(t < wid, s_t, 0)
        pltpu.sync_copy(tots_hbm.at[pl.ds(_al8(wid * NT), NT)], totv)

        def scan(i, c):
            x = totv[pl.ds(i * 16, 16)]
            inc = plsc.cumsum(x)
            pfxv[pl.ds(i * 16, 16)] = c + inc - x
            return c + inc[15]

        c_end = lax.fori_loop(0, NGRP, scan, carry)
        pltpu.sync_copy(pfxv, pfx_hbm.at[pl.ds(_al8(wid * NT), NT)])

        @pl.when(wid == 31)
        def _():
            padv[pl.ds(0, 16)] = jnp.broadcast_to(c_end, (16,))
            pltpu.sync_copy(padv, pfx_hbm.at[pl.ds(NPAD, 16)])

        def cpy(i, c):
            runv[pl.ds(i * 16, 16)] = pfxv[pl.ds(i * 16, 16)]
            return c

        lax.fori_loop(0, NGRP, cpy, 0)
        for t in range(32):
            pltpu.sync_copy(runv, starts_hbm.at[pl.ds(_al8(t * NPAD + wid * NT), NT)])
            if t < 31:
                pltpu.sync_copy(hists_hbm.at[pl.ds(_al8(t * NPAD + wid * NT), NT)], hrow)

                def add(i, c):
                    runv[pl.ds(i * 16, 16)] = (
                        runv[pl.ds(i * 16, 16)] + hrow[pl.ds(i * 16, 16)]
                    )
                    return c

                lax.fori_loop(0, NGRP, add, 0)

    return k


_sc_offsets = _make_sc_offsets()


def _make_sc_permute():
    @functools.partial(
        pl.kernel,
        mesh=plsc.VectorSubcoreMesh(**_SC_MESH),
        compiler_params=_SC_PARAMS,
        out_type=[
            jax.ShapeDtypeStruct((EPAD,), jnp.int32),
            jax.ShapeDtypeStruct((EPAD,), jnp.float32),
        ],
        scratch_types=[
            pltpu.VMEM((NPAD,), jnp.int32),
            pltpu.VMEM((WIN,), jnp.int32),
            pltpu.VMEM((WIN,), jnp.int32),
            pltpu.VMEM((WIN,), jnp.float32),
            pltpu.VMEM((WIN,), jnp.int32),
            pltpu.VMEM((WIN,), jnp.int32),
            pltpu.VMEM((WIN,), jnp.float32),
            pltpu.VMEM((512,), jnp.int32),
            pltpu.VMEM((512,), jnp.float32),
            pltpu.SemaphoreType.DMA,
        ],
    )
    def k(dst_hbm, srcin_hbm, attr_hbm, starts_hbm, osrc_hbm, oattr_hbm,
          ctr_v, dst_v, src_v, attr_v, slot_v, vsrc_v, vattr_v, zi_v, zf_v, sem):
        wid = _wid()
        iota = lax.iota(jnp.int32, 16)
        pltpu.sync_copy(starts_hbm.at[pl.ds(_al8(wid * NPAD), NPAD)], ctr_v)

        @pl.when(wid == 0)
        def _():
            def z(i, c):
                zi_v[pl.ds(i * 16, 16)] = jnp.zeros((16,), jnp.int32)
                zf_v[pl.ds(i * 16, 16)] = jnp.zeros((16,), jnp.float32)
                return c

            lax.fori_loop(0, 32, z, 0)
            pltpu.sync_copy(zi_v, osrc_hbm.at[pl.ds(E, 512)])
            pltpu.sync_copy(zf_v, oattr_hbm.at[pl.ds(E, 512)])

        def window(w, c):
            off = wid * EW + w * WIN
            pltpu.sync_copy(dst_hbm.at[pl.ds(_al8(off), WIN)], dst_v)
            pltpu.sync_copy(srcin_hbm.at[pl.ds(_al8(off), WIN)], src_v)
            pltpu.sync_copy(attr_hbm.at[pl.ds(_al8(off), WIN)], attr_v)

            def blk(i, c2):
                d16 = dst_v[pl.ds(i * 16, 16)]
                s16 = src_v[pl.ds(i * 16, 16)]
                a16 = attr_v[pl.ds(i * 16, 16)]
                sk, lane = plsc.sort_key_val(d16, iota)
                prev = sk.at[jnp.maximum(iota - 1, 0)].get(mode="promise_in_bounds")
                head = (sk != prev) | (iota == 0)
                hpos = plsc.cummax(jnp.where(head, iota, -1))
                rank = iota - hpos
                nxt = sk.at[jnp.minimum(iota + 1, 15)].get(mode="promise_in_bounds")
                tail = (sk != nxt) | (iota == 15)
                base = plsc.load_gather(ctr_v, [sk])
                slot = base + rank
                plsc.store_scatter(ctr_v, [sk], slot + 1, mask=tail)
                slot_v[pl.ds(i * 16, 16)] = slot
                vsrc_v[pl.ds(i * 16, 16)] = s16.at[lane].get(mode="promise_in_bounds")
                vattr_v[pl.ds(i * 16, 16)] = a16.at[lane].get(mode="promise_in_bounds")
                return c2

            lax.fori_loop(0, WIN // 16, blk, 0)
            pltpu.async_copy(vsrc_v, osrc_hbm.at[slot_v], sem).wait()
            pltpu.async_copy(vattr_v, oattr_hbm.at[slot_v], sem).wait()
            return c

        lax.fori_loop(0, NWIN, window, 0)

    return k


_sc_permute = _make_sc_permute()


def _make_sc_layer():
    @functools.partial(
        pl.kernel,
        mesh=plsc.VectorSubcoreMesh(**_SC_MESH),
        compiler_params=_SC_PARAMS,
        out_type=jax.ShapeDtypeStruct((NPAD * 128,), jnp.float32),
        scratch_types=[
            pltpu.VMEM((NT + 16,), jnp.int32),
            pltpu.VMEM((G,), jnp.int32),
            pltpu.VMEM((G,), jnp.float32),
            pltpu.VMEM((G, 128), jnp.float32),
            pltpu.VMEM((2048,), jnp.float32),
            pltpu.VMEM((32,), jnp.float32),
            pltpu.SemaphoreType.DMA,
        ],
    )
    def k(btab_hbm, src_hbm, attr_hbm, pfx_hbm, p_hbm, accs_hbm,
          pfx_v, src_v, attr_v, rows_v, outg_v, p_v, sem):
        wid = _wid()
        pltpu.sync_copy(pfx_hbm.at[pl.ds(_al8(wid * NT), NT + 16)], pfx_v)
        pltpu.sync_copy(p_hbm, p_v)
        p0 = p_v[pl.ds(0, 16)]
        p1 = p_v[pl.ds(16, 16)]
        inf = jnp.float32(jnp.inf)

        def init_acc():
            return (
                jnp.zeros((16,), jnp.float32), jnp.zeros((16,), jnp.float32),
                jnp.zeros((16,), jnp.float32), jnp.zeros((16,), jnp.float32),
                jnp.full((16,), inf, jnp.float32), jnp.full((16,), inf, jnp.float32),
                jnp.full((16,), -inf, jnp.float32), jnp.full((16,), -inf, jnp.float32),
            )

        def edge_body(r, acc):
            s0, s1, q0, q1, n0, n1, x0, x1 = acc
            b0 = rows_v[r, pl.ds(0, 16)]
            b1 = rows_v[r, pl.ds(16, 16)]
            av = plsc.load_gather(attr_v, [jnp.full((16,), 1, jnp.int32) * r])
            v0 = b0 + av * p0
            v1 = b1 + av * p1
            return (s0 + v0, s1 + v1, q0 + v0 * v0, q1 + v1 * v1,
                    jnp.minimum(n0, v0), jnp.minimum(n1, v1),
                    jnp.maximum(x0, v0), jnp.maximum(x1, v1))

        def flush(j, acc):
            s0, s1, q0, q1, n0, n1, x0, x1 = acc
            o = j * 128
            outg_v[pl.ds(o, 16)] = s0
            outg_v[pl.ds(o + 16, 16)] = s1
            outg_v[pl.ds(o + 32, 16)] = q0
            outg_v[pl.ds(o + 48, 16)] = q1
            outg_v[pl.ds(o + 64, 16)] = n0
            outg_v[pl.ds(o + 80, 16)] = n1
            outg_v[pl.ds(o + 96, 16)] = x0
            outg_v[pl.ds(o + 112, 16)] = x1

        def stage(base_al):
            pltpu.sync_copy(src_hbm.at[pl.ds(_al8(base_al), G)], src_v)
            pltpu.sync_copy(attr_hbm.at[pl.ds(_al8(base_al), G)], attr_v)
            pltpu.async_copy(btab_hbm.at[src_v], rows_v, sem).wait()

        def group(g, c):
            pfx16 = pfx_v[pl.ds(g * 16, 16)]
            pfx16n = pfx_v[pl.ds(g * 16 + 16, 16)]
            eb = pfx16[0]
            eend = pfx16n[0]
            eb_al = (eb >> 3) << 3
            fast = (eend - eb_al) <= G

            def fast_fn():
                stage(eb_al)
                for j in range(16):
                    st = pfx16[j]
                    en = pfx16[j + 1] if j < 15 else pfx16n[0]
                    acc = lax.fori_loop(st - eb_al, en - eb_al, edge_body,
                                        init_acc())
                    flush(j, acc)

            def slow_fn():
                for j in range(16):
                    st = pfx16[j]
                    en = pfx16[j + 1] if j < 15 else pfx16n[0]
                    nch = (en - st + 399) // 400

                    def chunk(ci, acc):
                        cb = st + ci * 400
                        cb_al = (cb >> 3) << 3
                        cend = jnp.minimum(en, cb + 400)
                        stage(cb_al)
                        return lax.fori_loop(cb - cb_al, cend - cb_al,
                                             edge_body, acc)

                    acc = lax.fori_loop(0, nch, chunk, init_acc())
                    flush(j, acc)

            lax.cond(fast, fast_fn, slow_fn)
            pltpu.sync_copy(
                outg_v, accs_hbm.at[pl.ds(_al8((wid * NT + g * 16) * 128), 2048)])
            return c

        lax.fori_loop(0, NGRP, group, 0)

    return k


_sc_layer = _make_sc_layer()


def _full(shape):
    return pl.BlockSpec(shape, lambda b: tuple(0 for _ in shape))


def _tc_prep_body(x_ref, wa_ref, wb_ref, qc_ref, btab_ref, astar_ref):
    xb = x_ref[...]
    hi = jax.lax.Precision.HIGHEST
    bt = jnp.dot(xb, wb_ref[...], precision=hi)
    btab_ref[...] = jnp.concatenate(
        [bt, jnp.zeros((xb.shape[0], 96), jnp.float32)], axis=1)
    astar_ref[...] = jnp.dot(xb, wa_ref[...], precision=hi) + qc_ref[...]


def _tc_prep(xp, wa, wb, qc):
    return pl.pallas_call(
        _tc_prep_body,
        grid=(NB,),
        in_specs=[
            pl.BlockSpec((1024, F), lambda b: (b, 0)),
            _full((F, 32)),
            _full((F, 32)),
            _full((1, 32)),
        ],
        out_specs=[
            pl.BlockSpec((1024, 128), lambda b: (b, 0)),
            pl.BlockSpec((1024, 32), lambda b: (b, 0)),
        ],
        out_shape=[
            jax.ShapeDtypeStruct((NPAD, 128), jnp.float32),
            jax.ShapeDtypeStruct((NPAD, 32), jnp.float32),
        ],
    )(xp, wa, wb, qc)


def _tc_combine_body(accs_ref, cnt_ref, astar_ref, x_ref, pwx_ref, wmain_ref,
                     wamp_ref, watt_ref, pb_ref, linw_ref, linb_ref,
                     y_ref, st_ref):
    hi = jax.lax.Precision.HIGHEST
    b = pl.program_id(0)
    accs = accs_ref[...]
    c2 = cnt_ref[...]
    Ap = astar_ref[...]
    xb = x_ref[...]
    deg = jnp.maximum(c2, 1.0)
    has = c2 > 0.0
    S = accs[:, 0:32]
    Qs = accs[:, 32:64]
    Nv = accs[:, 64:96]
    Xv = accs[:, 96:128]
    mean = (c2 * Ap + S) / deg
    msq = (c2 * Ap * Ap + 2.0 * Ap * S + Qs) / deg
    std = jnp.sqrt(jax.nn.relu(msq - mean * mean) + 1e-5)
    mn = jnp.where(has, Ap + Nv, 0.0)
    mx = jnp.where(has, Ap + Xv, 0.0)
    dlog = jnp.log(deg + 1.0)
    s1 = dlog / AVG_LOG
    s2 = AVG_LOG / dlog
    out = jnp.dot(xb, pwx_ref[...], precision=hi) + pb_ref[...]
    aggs = (mean, mn, mx, std)
    for kk in range(4):
        a25 = aggs[kk][:, :25]
        out = out + jnp.dot(a25, wmain_ref[...][kk], precision=hi)
        out = out + jnp.dot(a25, wamp_ref[...][kk], precision=hi) * s1
        out = out + jnp.dot(a25, watt_ref[...][kk], precision=hi) * s2
    y = jnp.dot(out, linw_ref[...], precision=hi) + linb_ref[...]
    y_ref[...] = y
    gid = b * 1024 + lax.broadcasted_iota(jnp.int32, (1024, 1), 0)
    ym = jnp.where(gid < N, y, 0.0)
    ssum = jnp.sum(ym, axis=0, keepdims=True)
    ssq = jnp.sum(ym * ym, axis=0, keepdims=True)
    row = jnp.concatenate([ssum, ssq, jnp.zeros((1, 6), jnp.float32)], axis=1)
    st_ref[...] = row[None]


def _tc_combine(accs, cnt2, astar, xp, pwx, wmain, wamp, watt, pb5, linw, linb):
    return pl.pallas_call(
        _tc_combine_body,
        grid=(NB,),
        in_specs=[
            pl.BlockSpec((1024, 128), lambda b: (b, 0)),
            pl.BlockSpec((1024, 1), lambda b: (b, 0)),
            pl.BlockSpec((1024, 32), lambda b: (b, 0)),
            pl.BlockSpec((1024, F), lambda b: (b, 0)),
            _full((F, F)),
            _full((4, 25, F)),
            _full((4, 25, F)),
            _full((4, 25, F)),
            _full((1, F)),
            _full((F, F)),
            _full((1, F)),
        ],
        out_specs=[
            pl.BlockSpec((1024, F), lambda b: (b, 0)),
            pl.BlockSpec((1, 1, 16), lambda b: (b, 0, 0)),
        ],
        out_shape=[
            jax.ShapeDtypeStruct((NPAD, F), jnp.float32),
            jax.ShapeDtypeStruct((NB, 1, 16), jnp.float32),
        ],
    )(accs, cnt2, astar, xp, pwx, wmain, wamp, watt, pb5, linw, linb)


def _bn_scale(st_ref, g_ref, b_ref):
    st = st_ref[...]
    mu = jnp.sum(st[:, 0, 0:F], axis=0, keepdims=True) / N
    var = jnp.sum(st[:, 0, 5:10], axis=0, keepdims=True) / N - mu * mu
    sc = g_ref[...] / jnp.sqrt(var + 1e-5)
    return mu, sc, b_ref[...]


def _tc_finish0_body(y_ref, st_ref, g_ref, b_ref, wa_ref, wb_ref, qc_ref,
                     h_ref, btab_ref, astar_ref):
    hi = jax.lax.Precision.HIGHEST
    mu, sc, bb = _bn_scale(st_ref, g_ref, b_ref)
    h = jax.nn.relu((y_ref[...] - mu) * sc + bb)
    h_ref[...] = h
    bt = jnp.dot(h, wb_ref[...], precision=hi)
    btab_ref[...] = jnp.concatenate(
        [bt, jnp.zeros((h.shape[0], 96), jnp.float32)], axis=1)
    astar_ref[...] = jnp.dot(h, wa_ref[...], precision=hi) + qc_ref[...]


def _tc_finish0(y, st, g, b, wa, wb, qc):
    return pl.pallas_call(
        _tc_finish0_body,
        grid=(NB,),
        in_specs=[
            pl.BlockSpec((1024, F), lambda b: (b, 0)),
            _full((NB, 1, 16)),
            _full((1, F)),
            _full((1, F)),
            _full((F, 32)),
            _full((F, 32)),
            _full((1, 32)),
        ],
        out_specs=[
            pl.BlockSpec((1024, F), lambda b: (b, 0)),
            pl.BlockSpec((1024, 128), lambda b: (b, 0)),
            pl.BlockSpec((1024, 32), lambda b: (b, 0)),
        ],
        out_shape=[
            jax.ShapeDtypeStruct((NPAD, F), jnp.float32),
            jax.ShapeDtypeStruct((NPAD, 128), jnp.float32),
            jax.ShapeDtypeStruct((NPAD, 32), jnp.float32),
        ],
    )(y, st, g, b, wa, wb, qc)


def _tc_finish1_body(y_ref, st_ref, g_ref, b_ref, h_ref):
    mu, sc, bb = _bn_scale(st_ref, g_ref, b_ref)
    h_ref[...] = jax.nn.relu((y_ref[...] - mu) * sc + bb)


def _tc_finish1(y, st, g, b):
    return pl.pallas_call(
        _tc_finish1_body,
        grid=(NB,),
        in_specs=[
            pl.BlockSpec((1024, F), lambda b: (b, 0)),
            _full((NB, 1, 16)),
            _full((1, F)),
            _full((1, F)),
        ],
        out_specs=pl.BlockSpec((1024, F), lambda b: (b, 0)),
        out_shape=jax.ShapeDtypeStruct((NPAD, F), jnp.float32),
    )(y, st, g, b)


def _tc_pool_body(h_ref, batch_ref, w1_ref, b1_ref, w2_ref, b2_ref,
                  w3_ref, b3_ref, out_ref, acc_ref):
    hi = jax.lax.Precision.HIGHEST
    b = pl.program_id(0)

    @pl.when(b == 0)
    def _():
        acc_ref[...] = jnp.zeros_like(acc_ref)

    bt = batch_ref[0, 0, :][:, None]
    gi = lax.broadcasted_iota(jnp.int32, (1, NGRAPHS), 1)
    oh = (bt == gi).astype(jnp.float32)
    contrib = lax.dot_general(oh, h_ref[...], (((0,), (0,)), ((), ())),
                              precision=hi)
    acc_ref[:, 0:F] = acc_ref[:, 0:F] + contrib

    @pl.when(b == NB - 1)
    def _():
        g = acc_ref[:, 0:F]
        z = jax.nn.relu(jnp.dot(g, w1_ref[...], precision=hi) + b1_ref[...])
        z = jax.nn.relu(jnp.dot(z, w2_ref[...], precision=hi) + b2_ref[...])
        out_ref[...] = jnp.dot(z, w3_ref[...], precision=hi) + b3_ref[...]


def _tc_pool(h2, batch3, w1, b1, w2, b2, w3, b3):
    return pl.pallas_call(
        _tc_pool_body,
        grid=(NB,),
        in_specs=[
            pl.BlockSpec((1024, F), lambda b: (b, 0)),
            pl.BlockSpec((1, 1, 1024), lambda b: (b, 0, 0)),
            _full((F, F)),
            _full((1, F)),
            _full((F, 10)),
            _full((1, 10)),
            _full((10, 1)),
            _full((1, 1)),
        ],
        out_specs=pl.BlockSpec((NGRAPHS, 1), lambda b: (0, 0)),
        out_shape=jax.ShapeDtypeStruct((NGRAPHS, 1), jnp.float32),
        scratch_shapes=[pltpu.VMEM((NGRAPHS, 8), jnp.float32)],
    )(h2, batch3, w1, b1, w2, b2, w3, b3)


def _prep_conv_weights(eW, eb, preW, preb, postW, postb):
    wa = preW[:, :F, :].transpose(1, 0, 2).reshape(F, T * F)
    wb = preW[:, F:2 * F, :].transpose(1, 0, 2).reshape(F, T * F)
    wc = preW[:, 2 * F:, :]
    P = jnp.einsum('f,tfo->to', eW[0], wc).reshape(T * F)
    Q = jnp.einsum('f,tfo->to', eb, wc).reshape(T * F)
    qc = Q + preb.reshape(T * F)
    pad = lambda m: jnp.concatenate(
        [m, jnp.zeros(m.shape[:-1] + (32 - m.shape[-1],), jnp.float32)], -1)
    wa32, wb32 = pad(wa), pad(wb)
    qc32 = pad(qc[None, :])
    p32 = pad(P[None, :])[0]
    pwx = postW[:, :F, 0].T
    eye = jnp.eye(T, dtype=jnp.float32)
    def blockdiag(col0):
        base = postW[:, col0:col0 + 20, 0].reshape(T, 4, F)
        return jnp.einsum('tko,tu->ktou', base, eye).reshape(4, T * F, T)
    wmain = blockdiag(F)
    wamp = blockdiag(F + 20)
    watt = blockdiag(F + 40)
    pb5 = postb[:, 0][None, :]
    return wa32, wb32, qc32, p32, pwx, wmain, wamp, watt, pb5


def kernel(x, edge_index, edge_attr, batch,
           c0_edge_W, c0_edge_b, c0_pre_W, c0_pre_b, c0_post_W, c0_post_b,
           c0_lin_W, c0_lin_b, bn0_g, bn0_b,
           c1_edge_W, c1_edge_b, c1_pre_W, c1_pre_b, c1_post_W, c1_post_b,
           c1_lin_W, c1_lin_b, bn1_g, bn1_b,
           mlp_W1, mlp_b1, mlp_W2, mlp_b2, mlp_W3, mlp_b3):
    xp = jnp.concatenate(
        [x, jnp.zeros((NPAD - N, F), jnp.float32)], axis=0)
    attr_flat = edge_attr[:, 0]
    batch_pad = jnp.concatenate(
        [batch, jnp.full((NPAD - N,), NGRAPHS, jnp.int32)]).reshape(NB, 1, 1024)

    w0 = _prep_conv_weights(c0_edge_W, c0_edge_b, c0_pre_W, c0_pre_b,
                            c0_post_W, c0_post_b)
    w1 = _prep_conv_weights(c1_edge_W, c1_edge_b, c1_pre_W, c1_pre_b,
                            c1_post_W, c1_post_b)
    wa0, wb0, qc0, p0, pwx0, wmain0, wamp0, watt0, pb0 = w0
    wa1, wb1, qc1, p1, pwx1, wmain1, wamp1, watt1, pb1 = w1

    src_arr = edge_index[0]
    dst_arr = edge_index[1]
    hists = _sc_hist(dst_arr)
    tots, cntf, ss = _sc_colsum(hists)
    prefix, starts = _sc_offsets(ss, tots, hists)
    ssrc, sattr = _sc_permute(dst_arr, src_arr, attr_flat, starts)
    cnt2 = cntf.reshape(NPAD, 1)

    btab0, astar0 = _tc_prep(xp, wa0, wb0, qc0)
    accs0 = _sc_layer(btab0, ssrc, sattr, prefix, p0).reshape(NPAD, 128)
    y0, st0 = _tc_combine(accs0, cnt2, astar0, xp, pwx0, wmain0, wamp0,
                          watt0, pb0, c0_lin_W, c0_lin_b[None, :])
    h0, btab1, astar1 = _tc_finish0(y0, st0, bn0_g[None, :], bn0_b[None, :],
                                    wa1, wb1, qc1)

    accs1 = _sc_layer(btab1, ssrc, sattr, prefix, p1).reshape(NPAD, 128)
    y1, st1 = _tc_combine(accs1, cnt2, astar1, h0, pwx1, wmain1, wamp1,
                          watt1, pb1, c1_lin_W, c1_lin_b[None, :])
    h1 = _tc_finish1(y1, st1, bn1_g[None, :], bn1_b[None, :])

    return _tc_pool(h1, batch_pad, mlp_W1, mlp_b1[None, :], mlp_W2,
                    mlp_b2[None, :], mlp_W3, mlp_b3[None, :])
